# baseline (speedup 1.0000x reference)
import functools

import jax
import jax.numpy as jnp
from jax import lax
from jax.experimental import pallas as pl
from jax.experimental.pallas import tpu as pltpu
from jax.experimental.pallas import tpu_sc as plsc

N = 10000
E = 320000
D = 128
EPS = 1e-5

NC = 2
NS = 16
NW = NC * NS
CHUNK = 128
PER_W = 78 * CHUNK
TAIL0 = NW * PER_W
NJ = 79
NB = 3
NBG = 4
ROWS_PER_TILE = 624


def _pack_half(t):
    b = jax.lax.bitcast_convert_type(t.astype(jnp.bfloat16), jnp.uint16)
    lo = b[:, :D // 2].astype(jnp.uint32)
    hi = b[:, D // 2:].astype(jnp.uint32) << 16
    return jax.lax.bitcast_convert_type(lo | hi, jnp.int32)


def _mesh():
    return plsc.VectorSubcoreMesh(
        core_axis_name="c", subcore_axis_name="s", num_cores=NC,
        num_subcores=NS)


def _off(wid, t):
    t = jnp.clip(t, 0, NJ - 1)
    return jnp.where(t < NJ - 1, wid * PER_W + t * CHUNK,
                     TAIL0 + wid * CHUNK)


def _valid(wid, t):
    return (t < NJ - 1) & (t >= 0) | ((t == NJ - 1) & (wid < 4))


def _sc_segment_sum(recv2d, ef):
    NBK = 2
    WPW = 80
    IBATCH = 16

    def body(recv_hbm, ef_hbm, out0_hbm, out1_hbm, acc_sh, idxb_v, rows_v,
             semI, semR, semS):
        c = lax.axis_index("c")
        s = lax.axis_index("s")
        wid = s * NC + c
        nwin = jnp.where(wid < NW - 1, WPW, 2500 - (NW - 1) * WPW)

        zero16 = jnp.zeros((16,), jnp.float32)

        def zrow(i, _):
            r = i // (D // 16)
            k = i % (D // 16)
            rows_v[0, r, pl.ds(k * 16, 16)] = zero16
            return 0

        lax.fori_loop(0, CHUNK * (D // 16), zrow, 0)
        zbase = s * ROWS_PER_TILE
        for i in range(4):
            pltpu.sync_copy(rows_v.at[0],
                            acc_sh.at[pl.ds(zbase + i * CHUNK, CHUNK)])
        pltpu.sync_copy(rows_v.at[0, pl.ds(0, 112)],
                        acc_sh.at[pl.ds(zbase + 4 * CHUNK, 112)])

        @pl.when(s == NS - 1)
        def _():
            pltpu.sync_copy(rows_v.at[0, pl.ds(0, 16)],
                            acc_sh.at[pl.ds(NS * ROWS_PER_TILE, 16)])

        plsc.subcore_barrier()

        def valid1(t):
            return (t >= 0) & (t < nwin)

        def start_batch(m):
            pltpu.async_copy(
                recv_hbm.at[pl.ds(WPW * wid + m * IBATCH, IBATCH)],
                idxb_v.at[m % 2], semI)

        def wait_batch(m):
            pltpu.make_async_copy(
                recv_hbm.at[pl.ds(WPW * wid + m * IBATCH, IBATCH)],
                idxb_v.at[m % 2], semI).wait()

        def start_rows(t):
            b = t % NBK
            g = WPW * wid + jnp.clip(t, 0, WPW - 1)
            pltpu.async_copy(ef_hbm.at[pl.ds(g * CHUNK, CHUNK)],
                             rows_v.at[b], semR)

        def wait_rows(t):
            b = t % NBK
            g = WPW * wid + jnp.clip(t, 0, WPW - 1)
            pltpu.make_async_copy(ef_hbm.at[pl.ds(g * CHUNK, CHUNK)],
                                  rows_v.at[b], semR).wait()

        def idx_row(t):
            return idxb_v.at[(t // IBATCH) % 2, t % IBATCH]

        def start_scatter(t):
            b = t % NBK
            pltpu.async_copy(rows_v.at[b], acc_sh.at[idx_row(t)], semS,
                             add=True)

        def wait_scatter(t):
            b = t % NBK
            pltpu.make_async_copy(rows_v.at[b], acc_sh.at[idx_row(t)],
                                  semS).wait()

        start_batch(0)
        start_rows(0)

        def step(t, _):
            @pl.when((t % IBATCH == 0) & valid1(t))
            def _():
                wait_batch(t // IBATCH)

            @pl.when(valid1(t))
            def _():
                wait_rows(t)
                start_scatter(t)

            @pl.when(valid1(t - (NBK - 1)))
            def _():
                wait_scatter(t - (NBK - 1))

            @pl.when(valid1(t + 1))
            def _():
                start_rows(t + 1)

            m_next = t // IBATCH + 1
            @pl.when((t % IBATCH == IBATCH - 3)
                     & (m_next * IBATCH < nwin))
            def _():
                start_batch(m_next)

            return 0

        lax.fori_loop(0, WPW, step, 0)

        @pl.when(nwin == WPW)
        def _():
            wait_scatter(WPW - 1)

        plsc.subcore_barrier()

        def dump(out_hbm):
            pltpu.sync_copy(acc_sh.at[pl.ds(zbase, ROWS_PER_TILE)],
                            out_hbm.at[pl.ds(zbase, ROWS_PER_TILE)])

            @pl.when(s == NS - 1)
            def _():
                pltpu.sync_copy(
                    acc_sh.at[pl.ds(NS * ROWS_PER_TILE, 16)],
                    out_hbm.at[pl.ds(NS * ROWS_PER_TILE, 16)])

        @pl.when(c == 0)
        def _():
            dump(out0_hbm)

        @pl.when(c == 1)
        def _():
            dump(out1_hbm)

    f = pl.kernel(
        body,
        out_type=(jax.ShapeDtypeStruct((N, D), jnp.float32),
                  jax.ShapeDtypeStruct((N, D), jnp.float32)),
        mesh=_mesh(),
        scratch_types=[
            pltpu.VMEM_SHARED((N, D), jnp.float32),
            pltpu.VMEM((2, IBATCH, CHUNK), jnp.int32),
            pltpu.VMEM((NBK, CHUNK, D), jnp.float32),
            pltpu.SemaphoreType.DMA,
            pltpu.SemaphoreType.DMA,
            pltpu.SemaphoreType.DMA,
        ],
    )
    return f(recv2d, ef)


def _sc_gather2(tA, tB, send1d, recv1d, base, nE):
    nwin = nE // CHUNK
    NJ2 = (nwin + NS - 1) // NS
    rem = nwin - (NJ2 - 1) * NS

    def valid2(s, j):
        return (j < NJ2 - 1) & (j >= 0) | ((j == NJ2 - 1) & (s < rem))

    def off2(s, j):
        j = jnp.clip(j, 0, NJ2 - 1)
        return (j * NS + s) * CHUNK

    def body(tA_hbm, tB_hbm, send_hbm, recv_hbm, outA_hbm, outB_hbm,
             tab_sh, idx_v, rows_v, semI, semG, semO):
        c = lax.axis_index("c")
        s = lax.axis_index("s")

        zbase = s * ROWS_PER_TILE

        def stage(tab_hbm):
            pltpu.sync_copy(tab_hbm.at[pl.ds(zbase, ROWS_PER_TILE)],
                            tab_sh.at[pl.ds(zbase, ROWS_PER_TILE)])

            @pl.when(s == NS - 1)
            def _():
                pltpu.sync_copy(tab_hbm.at[pl.ds(NS * ROWS_PER_TILE, 16)],
                                tab_sh.at[pl.ds(NS * ROWS_PER_TILE, 16)])

        @pl.when(c == 0)
        def _():
            stage(tA_hbm)

        @pl.when(c == 1)
        def _():
            stage(tB_hbm)

        plsc.subcore_barrier()

        def start_idx(idx_hbm, j):
            b = j % NB
            o = off2(s, j)
            pltpu.async_copy(idx_hbm.at[pl.ds(base + o, CHUNK)],
                             idx_v.at[b], semI)

        def wait_idx(idx_hbm, j):
            b = j % NB
            o = off2(s, j)
            pltpu.make_async_copy(idx_hbm.at[pl.ds(base + o, CHUNK)],
                                  idx_v.at[b], semI).wait()

        def start_gather(j):
            b = j % NB
            pltpu.async_copy(tab_sh.at[idx_v.at[b]], rows_v.at[b], semG)

        def wait_gather(j):
            b = j % NB
            pltpu.make_async_copy(tab_sh.at[idx_v.at[b]], rows_v.at[b],
                                  semG).wait()

        def start_out(out_hbm, j):
            b = j % NB
            o = off2(s, j)
            pltpu.async_copy(rows_v.at[b], out_hbm.at[pl.ds(o, CHUNK)],
                             semO)

        def wait_out(out_hbm, j):
            b = j % NB
            o = off2(s, j)
            pltpu.make_async_copy(rows_v.at[b],
                                  out_hbm.at[pl.ds(o, CHUNK)],
                                  semO).wait()

        def run(idx_hbm, out_hbm):
            start_idx(idx_hbm, 0)

            def step(j, _):
                @pl.when(valid2(s, j - NB))
                def _():
                    wait_out(out_hbm, j - NB)

                @pl.when(valid2(s, j))
                def _():
                    wait_idx(idx_hbm, j)
                    start_gather(j)

                @pl.when(valid2(s, j + 1))
                def _():
                    start_idx(idx_hbm, j + 1)

                @pl.when(valid2(s, j))
                def _():
                    wait_gather(j)
                    start_out(out_hbm, j)

                return 0

            lax.fori_loop(0, NJ2, step, 0)
            for dt in range(NB):
                t = NJ2 - 1 - dt

                @pl.when(valid2(s, t))
                def _():
                    wait_out(out_hbm, t)

        @pl.when(c == 0)
        def _():
            run(send_hbm, outA_hbm)

        @pl.when(c == 1)
        def _():
            run(recv_hbm, outB_hbm)

    f = pl.kernel(
        body,
        out_type=(jax.ShapeDtypeStruct((nE, D), jnp.float32),
                  jax.ShapeDtypeStruct((nE, D), jnp.float32)),
        mesh=_mesh(),
        scratch_types=[
            pltpu.VMEM_SHARED((N, D), jnp.float32),
            pltpu.VMEM((NB, CHUNK), jnp.int32),
            pltpu.VMEM((NB, CHUNK, D), jnp.float32),
            pltpu.SemaphoreType.DMA,
            pltpu.SemaphoreType.DMA,
            pltpu.SemaphoreType.DMA,
        ],
    )
    return f(tA, tB, send1d, recv1d)


def _layer_norm(h, gamma, beta):
    mu = jnp.mean(h, axis=-1, keepdims=True)
    var = jnp.mean((h - mu) ** 2, axis=-1, keepdims=True)
    return (h - mu) * lax.rsqrt(var + EPS) * gamma + beta


def _tc_node_mlp(nodes, agg0, agg1, nW1a, nW1b, nb1, nW2, nb2, ng, nbeta,
                 WA, WB):
    BN = 1000

    def body(nodes_ref, a0_ref, a1_ref, nW1a_ref, nW1b_ref, nb1_ref,
             nW2_ref, nb2_ref, ng_ref, nbeta_ref, WA_ref, WB_ref,
             nf_ref, tA_ref, tB_ref):
        x = nodes_ref[...]
        a = a0_ref[...] + a1_ref[...]
        h = (jnp.dot(x, nW1a_ref[...], preferred_element_type=jnp.float32)
             + jnp.dot(a, nW1b_ref[...], preferred_element_type=jnp.float32)
             + nb1_ref[...])
        h = jnp.dot(h, nW2_ref[...],
                    preferred_element_type=jnp.float32) + nb2_ref[...]
        nf = _layer_norm(h, ng_ref[...], nbeta_ref[...])
        nf_ref[...] = nf
        tA_ref[...] = jnp.dot(nf, WA_ref[...],
                              preferred_element_type=jnp.float32)
        tB_ref[...] = jnp.dot(nf, WB_ref[...],
                              preferred_element_type=jnp.float32)

    row_spec = pl.BlockSpec((BN, D), lambda i: (i, 0))
    w_spec = pl.BlockSpec((D, D), lambda i: (0, 0))
    v_spec = pl.BlockSpec((D,), lambda i: (0,))
    return pl.pallas_call(
        body,
        grid=(N // BN,),
        in_specs=[row_spec, row_spec, row_spec, w_spec, w_spec, v_spec,
                  w_spec, v_spec, v_spec, v_spec, w_spec, w_spec],
        out_specs=[row_spec, row_spec, row_spec],
        out_shape=[jax.ShapeDtypeStruct((N, D), jnp.float32)] * 3,
    )(nodes, agg0, agg1, nW1a, nW1b, nb1, nW2, nb2, ng, nbeta, WA, WB)


def _tc_edge_pre(ef, Wc, bc):
    BE = 8000

    def body(ef_ref, Wc_ref, bc_ref, out_ref):
        out_ref[...] = jnp.dot(
            ef_ref[...], Wc_ref[...],
            preferred_element_type=jnp.float32) + bc_ref[...]

    return pl.pallas_call(
        body,
        grid=(E // BE,),
        in_specs=[pl.BlockSpec((BE, D), lambda i: (i, 0)),
                  pl.BlockSpec((D, D), lambda i: (0, 0)),
                  pl.BlockSpec((D,), lambda i: (0,))],
        out_specs=pl.BlockSpec((BE, D), lambda i: (i, 0)),
        out_shape=jax.ShapeDtypeStruct((E, D), jnp.float32),
    )(ef, Wc, bc)


def _tc_edge_mlp_strip(prev, srcsA, dstsB, efc, eg, ebeta,
                       block_off, n_rows):
    BE = 8000

    def body(*refs):
        if len(refs) == 6:
            sA_ref, dB_ref, efc_ref, eg_ref, ebeta_ref, out_ref = refs
        else:
            _, sA_ref, dB_ref, efc_ref, eg_ref, ebeta_ref, out_ref = refs
        h = sA_ref[...] + dB_ref[...] + efc_ref[...]
        out_ref[...] = _layer_norm(h, eg_ref[...], ebeta_ref[...])

    strip_spec = pl.BlockSpec((BE, D), lambda i: (i, 0))
    full_spec = pl.BlockSpec((BE, D), lambda i: (i + block_off, 0))
    v_spec = pl.BlockSpec((D,), lambda i: (0,))
    in_specs = [strip_spec, strip_spec, full_spec, v_spec, v_spec]
    args = (srcsA, dstsB, efc, eg, ebeta)
    aliases = {}
    if prev is not None:
        in_specs = [pl.BlockSpec(memory_space=pl.ANY)] + in_specs
        args = (prev,) + args
        aliases = {0: 0}
    return pl.pallas_call(
        body,
        grid=(n_rows // BE,),
        in_specs=in_specs,
        out_specs=full_spec,
        out_shape=jax.ShapeDtypeStruct((E, D), jnp.float32),
        input_output_aliases=aliases,
    )(*args)


def kernel(node_features, edge_features, senders, receivers,
           nW1, nb1, nW2, nb2, ng, nbeta,
           eW1, eb1, eW2, eb2, eg, ebeta):
    nodes = node_features[0]
    ef = edge_features[0]

    WA = eW1[:D] @ eW2
    WB = eW1[D:2 * D] @ eW2
    Wc = eW1[2 * D:] @ eW2
    bc = eb1 @ eW2 + eb2

    recv2d = jnp.pad(receivers.reshape(E // CHUNK, CHUNK),
                     ((0, 12), (0, 0)))
    efc = _tc_edge_pre(ef, Wc, bc)
    agg0, agg1 = _sc_segment_sum(recv2d, ef)
    nf, tA, tB = _tc_node_mlp(
        nodes, agg0, agg1, nW1[:D], nW1[D:], nb1, nW2, nb2, ng, nbeta,
        WA, WB)
    strips = (96000, 112000, 112000)
    gathered = []
    base = 0
    for nE in strips:
        gathered.append(_sc_gather2(tA, tB, senders, receivers, base, nE))
        base += nE
    ef_out = None
    base = 0
    for (sA, dB), nE in zip(gathered, strips):
        ef_out = _tc_edge_mlp_strip(ef_out, sA, dB, efc, eg,
                                    ebeta, base // 8000, nE)
        base += nE
    return (nf[None], ef_out[None])


# speedup vs baseline: 1.1858x; 1.1858x over previous
import functools

import jax
import jax.numpy as jnp
from jax import lax
from jax.experimental import pallas as pl
from jax.experimental.pallas import tpu as pltpu
from jax.experimental.pallas import tpu_sc as plsc

N = 10000
E = 320000
D = 128
EPS = 1e-5

NC = 2
NS = 16
NW = NC * NS
CHUNK = 128
PER_W = 78 * CHUNK
TAIL0 = NW * PER_W
NJ = 79
NB = 3
NBG = 4
ROWS_PER_TILE = 624


def _pack_half(t):
    b = jax.lax.bitcast_convert_type(t.astype(jnp.bfloat16), jnp.uint16)
    lo = b[:, :D // 2].astype(jnp.uint32)
    hi = b[:, D // 2:].astype(jnp.uint32) << 16
    return jax.lax.bitcast_convert_type(lo | hi, jnp.int32)


def _mesh():
    return plsc.VectorSubcoreMesh(
        core_axis_name="c", subcore_axis_name="s", num_cores=NC,
        num_subcores=NS)


def _off(wid, t):
    t = jnp.clip(t, 0, NJ - 1)
    return jnp.where(t < NJ - 1, wid * PER_W + t * CHUNK,
                     TAIL0 + wid * CHUNK)


def _valid(wid, t):
    return (t < NJ - 1) & (t >= 0) | ((t == NJ - 1) & (wid < 4))


def _sc_segment_sum(recv2d, ef):
    NBK = 2
    WPW = 80
    IBATCH = 16

    def body(recv_hbm, ef_hbm, out0_hbm, out1_hbm, acc_sh, idxb_v, rows_v,
             semI, semR, semS):
        c = lax.axis_index("c")
        s = lax.axis_index("s")
        wid = s * NC + c
        nwin = jnp.where(wid < NW - 1, WPW, 2500 - (NW - 1) * WPW)

        zero16 = jnp.zeros((16,), jnp.float32)

        def zrow(i, _):
            r = i // (D // 16)
            k = i % (D // 16)
            rows_v[0, r, pl.ds(k * 16, 16)] = zero16
            return 0

        lax.fori_loop(0, CHUNK * (D // 16), zrow, 0)
        zbase = s * ROWS_PER_TILE
        for i in range(4):
            pltpu.sync_copy(rows_v.at[0],
                            acc_sh.at[pl.ds(zbase + i * CHUNK, CHUNK)])
        pltpu.sync_copy(rows_v.at[0, pl.ds(0, 112)],
                        acc_sh.at[pl.ds(zbase + 4 * CHUNK, 112)])

        @pl.when(s == NS - 1)
        def _():
            pltpu.sync_copy(rows_v.at[0, pl.ds(0, 16)],
                            acc_sh.at[pl.ds(NS * ROWS_PER_TILE, 16)])

        plsc.subcore_barrier()

        def valid1(t):
            return (t >= 0) & (t < nwin)

        def start_batch(m):
            pltpu.async_copy(
                recv_hbm.at[pl.ds(WPW * wid + m * IBATCH, IBATCH)],
                idxb_v.at[m % 2], semI)

        def wait_batch(m):
            pltpu.make_async_copy(
                recv_hbm.at[pl.ds(WPW * wid + m * IBATCH, IBATCH)],
                idxb_v.at[m % 2], semI).wait()

        def start_rows(t):
            b = t % NBK
            g = WPW * wid + jnp.clip(t, 0, WPW - 1)
            pltpu.async_copy(ef_hbm.at[pl.ds(g * CHUNK, CHUNK)],
                             rows_v.at[b], semR)

        def wait_rows(t):
            b = t % NBK
            g = WPW * wid + jnp.clip(t, 0, WPW - 1)
            pltpu.make_async_copy(ef_hbm.at[pl.ds(g * CHUNK, CHUNK)],
                                  rows_v.at[b], semR).wait()

        def idx_row(t):
            return idxb_v.at[(t // IBATCH) % 2, t % IBATCH]

        def start_scatter(t):
            b = t % NBK
            pltpu.async_copy(rows_v.at[b], acc_sh.at[idx_row(t)], semS,
                             add=True)

        def wait_scatter(t):
            b = t % NBK
            pltpu.make_async_copy(rows_v.at[b], acc_sh.at[idx_row(t)],
                                  semS).wait()

        start_batch(0)
        start_rows(0)

        def step(t, _):
            @pl.when((t % IBATCH == 0) & valid1(t))
            def _():
                wait_batch(t // IBATCH)

            @pl.when(valid1(t))
            def _():
                wait_rows(t)
                start_scatter(t)

            @pl.when(valid1(t - (NBK - 1)))
            def _():
                wait_scatter(t - (NBK - 1))

            @pl.when(valid1(t + 1))
            def _():
                start_rows(t + 1)

            m_next = t // IBATCH + 1
            @pl.when((t % IBATCH == IBATCH - 3)
                     & (m_next * IBATCH < nwin))
            def _():
                start_batch(m_next)

            return 0

        lax.fori_loop(0, WPW, step, 0)

        @pl.when(nwin == WPW)
        def _():
            wait_scatter(WPW - 1)

        plsc.subcore_barrier()

        def dump(out_hbm):
            pltpu.sync_copy(acc_sh.at[pl.ds(zbase, ROWS_PER_TILE)],
                            out_hbm.at[pl.ds(zbase, ROWS_PER_TILE)])

            @pl.when(s == NS - 1)
            def _():
                pltpu.sync_copy(
                    acc_sh.at[pl.ds(NS * ROWS_PER_TILE, 16)],
                    out_hbm.at[pl.ds(NS * ROWS_PER_TILE, 16)])

        @pl.when(c == 0)
        def _():
            dump(out0_hbm)

        @pl.when(c == 1)
        def _():
            dump(out1_hbm)

    f = pl.kernel(
        body,
        out_type=(jax.ShapeDtypeStruct((N, D), jnp.float32),
                  jax.ShapeDtypeStruct((N, D), jnp.float32)),
        mesh=_mesh(),
        scratch_types=[
            pltpu.VMEM_SHARED((N, D), jnp.float32),
            pltpu.VMEM((2, IBATCH, CHUNK), jnp.int32),
            pltpu.VMEM((NBK, CHUNK, D), jnp.float32),
            pltpu.SemaphoreType.DMA,
            pltpu.SemaphoreType.DMA,
            pltpu.SemaphoreType.DMA,
        ],
    )
    return f(recv2d, ef)


def _sc_gather2(tA, tB, send1d, recv1d, base, nE):
    nwin = nE // CHUNK
    NJ2 = (nwin + NS - 1) // NS
    rem = nwin - (NJ2 - 1) * NS

    def valid2(s, j):
        return (j < NJ2 - 1) & (j >= 0) | ((j == NJ2 - 1) & (s < rem))

    def off2(s, j):
        j = jnp.clip(j, 0, NJ2 - 1)
        return (j * NS + s) * CHUNK

    def body(tA_hbm, tB_hbm, send_hbm, recv_hbm, outA_hbm, outB_hbm,
             tab_sh, idx_v, rows_v, semI, semG, semO):
        c = lax.axis_index("c")
        s = lax.axis_index("s")

        zbase = s * ROWS_PER_TILE

        def stage(tab_hbm):
            pltpu.sync_copy(tab_hbm.at[pl.ds(zbase, ROWS_PER_TILE)],
                            tab_sh.at[pl.ds(zbase, ROWS_PER_TILE)])

            @pl.when(s == NS - 1)
            def _():
                pltpu.sync_copy(tab_hbm.at[pl.ds(NS * ROWS_PER_TILE, 16)],
                                tab_sh.at[pl.ds(NS * ROWS_PER_TILE, 16)])

        @pl.when(c == 0)
        def _():
            stage(tA_hbm)

        @pl.when(c == 1)
        def _():
            stage(tB_hbm)

        plsc.subcore_barrier()

        def start_idx(idx_hbm, j):
            b = j % NB
            o = off2(s, j)
            pltpu.async_copy(idx_hbm.at[pl.ds(base + o, CHUNK)],
                             idx_v.at[b], semI)

        def wait_idx(idx_hbm, j):
            b = j % NB
            o = off2(s, j)
            pltpu.make_async_copy(idx_hbm.at[pl.ds(base + o, CHUNK)],
                                  idx_v.at[b], semI).wait()

        def start_gather(j):
            b = j % NB
            pltpu.async_copy(tab_sh.at[idx_v.at[b]], rows_v.at[b], semG)

        def wait_gather(j):
            b = j % NB
            pltpu.make_async_copy(tab_sh.at[idx_v.at[b]], rows_v.at[b],
                                  semG).wait()

        def start_out(out_hbm, j):
            b = j % NB
            o = off2(s, j)
            pltpu.async_copy(rows_v.at[b], out_hbm.at[pl.ds(o, CHUNK)],
                             semO)

        def wait_out(out_hbm, j):
            b = j % NB
            o = off2(s, j)
            pltpu.make_async_copy(rows_v.at[b],
                                  out_hbm.at[pl.ds(o, CHUNK)],
                                  semO).wait()

        def run(idx_hbm, out_hbm):
            start_idx(idx_hbm, 0)

            def step(j, _):
                @pl.when(valid2(s, j - NB))
                def _():
                    wait_out(out_hbm, j - NB)

                @pl.when(valid2(s, j))
                def _():
                    wait_idx(idx_hbm, j)
                    start_gather(j)

                @pl.when(valid2(s, j + 1))
                def _():
                    start_idx(idx_hbm, j + 1)

                @pl.when(valid2(s, j))
                def _():
                    wait_gather(j)
                    start_out(out_hbm, j)

                return 0

            lax.fori_loop(0, NJ2, step, 0)
            for dt in range(NB):
                t = NJ2 - 1 - dt

                @pl.when(valid2(s, t))
                def _():
                    wait_out(out_hbm, t)

        @pl.when(c == 0)
        def _():
            run(send_hbm, outA_hbm)

        @pl.when(c == 1)
        def _():
            run(recv_hbm, outB_hbm)

    f = pl.kernel(
        body,
        out_type=(jax.ShapeDtypeStruct((nE, D), jnp.float32),
                  jax.ShapeDtypeStruct((nE, D), jnp.float32)),
        mesh=_mesh(),
        scratch_types=[
            pltpu.VMEM_SHARED((N, D), jnp.float32),
            pltpu.VMEM((NB, CHUNK), jnp.int32),
            pltpu.VMEM((NB, CHUNK, D), jnp.float32),
            pltpu.SemaphoreType.DMA,
            pltpu.SemaphoreType.DMA,
            pltpu.SemaphoreType.DMA,
        ],
    )
    return f(tA, tB, send1d, recv1d)


def _layer_norm(h, gamma, beta):
    mu = jnp.mean(h, axis=-1, keepdims=True)
    var = jnp.mean((h - mu) ** 2, axis=-1, keepdims=True)
    return (h - mu) * lax.rsqrt(var + EPS) * gamma + beta


def _tc_node_mlp(nodes, agg0, agg1, nW1a, nW1b, nb1, nW2, nb2, ng, nbeta,
                 WA, WB):
    BN = 1000

    def body(nodes_ref, a0_ref, a1_ref, nW1a_ref, nW1b_ref, nb1_ref,
             nW2_ref, nb2_ref, ng_ref, nbeta_ref, WA_ref, WB_ref,
             nf_ref, tA_ref, tB_ref):
        x = nodes_ref[...]
        a = a0_ref[...] + a1_ref[...]
        h = (jnp.dot(x, nW1a_ref[...], preferred_element_type=jnp.float32)
             + jnp.dot(a, nW1b_ref[...], preferred_element_type=jnp.float32)
             + nb1_ref[...])
        h = jnp.dot(h, nW2_ref[...],
                    preferred_element_type=jnp.float32) + nb2_ref[...]
        nf = _layer_norm(h, ng_ref[...], nbeta_ref[...])
        nf_ref[...] = nf
        tA_ref[...] = jnp.dot(nf, WA_ref[...],
                              preferred_element_type=jnp.float32)
        tB_ref[...] = jnp.dot(nf, WB_ref[...],
                              preferred_element_type=jnp.float32)

    row_spec = pl.BlockSpec((BN, D), lambda i: (i, 0))
    w_spec = pl.BlockSpec((D, D), lambda i: (0, 0))
    v_spec = pl.BlockSpec((D,), lambda i: (0,))
    return pl.pallas_call(
        body,
        grid=(N // BN,),
        in_specs=[row_spec, row_spec, row_spec, w_spec, w_spec, v_spec,
                  w_spec, v_spec, v_spec, v_spec, w_spec, w_spec],
        out_specs=[row_spec, row_spec, row_spec],
        out_shape=[jax.ShapeDtypeStruct((N, D), jnp.float32)] * 3,
    )(nodes, agg0, agg1, nW1a, nW1b, nb1, nW2, nb2, ng, nbeta, WA, WB)


def _tc_edge_mlp_strip(prev, srcsA, dstsB, ef, Wc, bc, eg, ebeta,
                       block_off, n_rows):
    BE = 8000

    def body(*refs):
        if len(refs) == 8:
            sA_ref, dB_ref, ef_ref, Wc_ref, bc_ref, eg_ref, ebeta_ref, \
                out_ref = refs
        else:
            _, sA_ref, dB_ref, ef_ref, Wc_ref, bc_ref, eg_ref, \
                ebeta_ref, out_ref = refs
        h = (sA_ref[...] + dB_ref[...]
             + jnp.dot(ef_ref[...], Wc_ref[...],
                       preferred_element_type=jnp.float32) + bc_ref[...])
        out_ref[...] = _layer_norm(h, eg_ref[...], ebeta_ref[...])

    strip_spec = pl.BlockSpec((BE, D), lambda i: (i, 0))
    full_spec = pl.BlockSpec((BE, D), lambda i: (i + block_off, 0))
    w_spec = pl.BlockSpec((D, D), lambda i: (0, 0))
    v_spec = pl.BlockSpec((D,), lambda i: (0,))
    in_specs = [strip_spec, strip_spec, full_spec, w_spec, v_spec, v_spec,
                v_spec]
    args = (srcsA, dstsB, ef, Wc, bc, eg, ebeta)
    aliases = {}
    if prev is not None:
        in_specs = [pl.BlockSpec(memory_space=pl.ANY)] + in_specs
        args = (prev,) + args
        aliases = {0: 0}
    return pl.pallas_call(
        body,
        grid=(n_rows // BE,),
        in_specs=in_specs,
        out_specs=full_spec,
        out_shape=jax.ShapeDtypeStruct((E, D), jnp.float32),
        input_output_aliases=aliases,
    )(*args)


def kernel(node_features, edge_features, senders, receivers,
           nW1, nb1, nW2, nb2, ng, nbeta,
           eW1, eb1, eW2, eb2, eg, ebeta):
    nodes = node_features[0]
    ef = edge_features[0]

    WA = eW1[:D] @ eW2
    WB = eW1[D:2 * D] @ eW2
    Wc = eW1[2 * D:] @ eW2
    bc = eb1 @ eW2 + eb2

    recv2d = jnp.pad(receivers.reshape(E // CHUNK, CHUNK),
                     ((0, 12), (0, 0)))
    agg0, agg1 = _sc_segment_sum(recv2d, ef)
    nf, tA, tB = _tc_node_mlp(
        nodes, agg0, agg1, nW1[:D], nW1[D:], nb1, nW2, nb2, ng, nbeta,
        WA, WB)
    strips = (64000, 88000, 88000, 80000)
    gathered = []
    base = 0
    for nE in strips:
        gathered.append(_sc_gather2(tA, tB, senders, receivers, base, nE))
        base += nE
    ef_out = None
    base = 0
    for (sA, dB), nE in zip(gathered, strips):
        ef_out = _tc_edge_mlp_strip(ef_out, sA, dB, ef, Wc, bc, eg,
                                    ebeta, base // 8000, nE)
        base += nE
    return (nf[None], ef_out[None])


# speedup vs baseline: 1.1874x; 1.0013x over previous
import functools

import jax
import jax.numpy as jnp
from jax import lax
from jax.experimental import pallas as pl
from jax.experimental.pallas import tpu as pltpu
from jax.experimental.pallas import tpu_sc as plsc

N = 10000
E = 320000
D = 128
EPS = 1e-5

NC = 2
NS = 16
NW = NC * NS
CHUNK = 128
PER_W = 78 * CHUNK
TAIL0 = NW * PER_W
NJ = 79
NB = 3
NBG = 4
ROWS_PER_TILE = 624


def _pack_half(t):
    b = jax.lax.bitcast_convert_type(t.astype(jnp.bfloat16), jnp.uint16)
    lo = b[:, :D // 2].astype(jnp.uint32)
    hi = b[:, D // 2:].astype(jnp.uint32) << 16
    return jax.lax.bitcast_convert_type(lo | hi, jnp.int32)


def _mesh():
    return plsc.VectorSubcoreMesh(
        core_axis_name="c", subcore_axis_name="s", num_cores=NC,
        num_subcores=NS)


def _off(wid, t):
    t = jnp.clip(t, 0, NJ - 1)
    return jnp.where(t < NJ - 1, wid * PER_W + t * CHUNK,
                     TAIL0 + wid * CHUNK)


def _valid(wid, t):
    return (t < NJ - 1) & (t >= 0) | ((t == NJ - 1) & (wid < 4))


def _sc_segment_sum(recv2d, ef):
    NBK = 2
    WPW = 80
    IBATCH = 16

    def body(recv_hbm, ef_hbm, out0_hbm, out1_hbm, acc_sh, idxb_v, rows_v,
             semI, semR, semS):
        c = lax.axis_index("c")
        s = lax.axis_index("s")
        wid = s * NC + c
        nwin = jnp.where(wid < NW - 1, WPW, 2500 - (NW - 1) * WPW)

        zero16 = jnp.zeros((16,), jnp.float32)

        def zrow(i, _):
            r = i // (D // 16)
            k = i % (D // 16)
            rows_v[0, r, pl.ds(k * 16, 16)] = zero16
            return 0

        lax.fori_loop(0, CHUNK * (D // 16), zrow, 0)
        zbase = s * ROWS_PER_TILE
        for i in range(4):
            pltpu.sync_copy(rows_v.at[0],
                            acc_sh.at[pl.ds(zbase + i * CHUNK, CHUNK)])
        pltpu.sync_copy(rows_v.at[0, pl.ds(0, 112)],
                        acc_sh.at[pl.ds(zbase + 4 * CHUNK, 112)])

        @pl.when(s == NS - 1)
        def _():
            pltpu.sync_copy(rows_v.at[0, pl.ds(0, 16)],
                            acc_sh.at[pl.ds(NS * ROWS_PER_TILE, 16)])

        plsc.subcore_barrier()

        def valid1(t):
            return (t >= 0) & (t < nwin)

        def start_batch(m):
            pltpu.async_copy(
                recv_hbm.at[pl.ds(WPW * wid + m * IBATCH, IBATCH)],
                idxb_v.at[m % 2], semI)

        def wait_batch(m):
            pltpu.make_async_copy(
                recv_hbm.at[pl.ds(WPW * wid + m * IBATCH, IBATCH)],
                idxb_v.at[m % 2], semI).wait()

        def start_rows(t):
            b = t % NBK
            g = WPW * wid + jnp.clip(t, 0, WPW - 1)
            pltpu.async_copy(ef_hbm.at[pl.ds(g * CHUNK, CHUNK)],
                             rows_v.at[b], semR)

        def wait_rows(t):
            b = t % NBK
            g = WPW * wid + jnp.clip(t, 0, WPW - 1)
            pltpu.make_async_copy(ef_hbm.at[pl.ds(g * CHUNK, CHUNK)],
                                  rows_v.at[b], semR).wait()

        def idx_row(t):
            return idxb_v.at[(t // IBATCH) % 2, t % IBATCH]

        def start_scatter(t):
            b = t % NBK
            pltpu.async_copy(rows_v.at[b], acc_sh.at[idx_row(t)], semS,
                             add=True)

        def wait_scatter(t):
            b = t % NBK
            pltpu.make_async_copy(rows_v.at[b], acc_sh.at[idx_row(t)],
                                  semS).wait()

        start_batch(0)
        start_rows(0)

        def step(t, _):
            @pl.when((t % IBATCH == 0) & valid1(t))
            def _():
                wait_batch(t // IBATCH)

            @pl.when(valid1(t))
            def _():
                wait_rows(t)
                start_scatter(t)

            @pl.when(valid1(t - (NBK - 1)))
            def _():
                wait_scatter(t - (NBK - 1))

            @pl.when(valid1(t + 1))
            def _():
                start_rows(t + 1)

            m_next = t // IBATCH + 1
            @pl.when((t % IBATCH == IBATCH - 3)
                     & (m_next * IBATCH < nwin))
            def _():
                start_batch(m_next)

            return 0

        lax.fori_loop(0, WPW, step, 0)

        @pl.when(nwin == WPW)
        def _():
            wait_scatter(WPW - 1)

        plsc.subcore_barrier()

        def dump(out_hbm):
            pltpu.sync_copy(acc_sh.at[pl.ds(zbase, ROWS_PER_TILE)],
                            out_hbm.at[pl.ds(zbase, ROWS_PER_TILE)])

            @pl.when(s == NS - 1)
            def _():
                pltpu.sync_copy(
                    acc_sh.at[pl.ds(NS * ROWS_PER_TILE, 16)],
                    out_hbm.at[pl.ds(NS * ROWS_PER_TILE, 16)])

        @pl.when(c == 0)
        def _():
            dump(out0_hbm)

        @pl.when(c == 1)
        def _():
            dump(out1_hbm)

    f = pl.kernel(
        body,
        out_type=(jax.ShapeDtypeStruct((N, D), jnp.float32),
                  jax.ShapeDtypeStruct((N, D), jnp.float32)),
        mesh=_mesh(),
        scratch_types=[
            pltpu.VMEM_SHARED((N, D), jnp.float32),
            pltpu.VMEM((2, IBATCH, CHUNK), jnp.int32),
            pltpu.VMEM((NBK, CHUNK, D), jnp.float32),
            pltpu.SemaphoreType.DMA,
            pltpu.SemaphoreType.DMA,
            pltpu.SemaphoreType.DMA,
        ],
    )
    return f(recv2d, ef)


def _sc_gather2(tA, tB, send1d, recv1d, base, nE):
    nwin = nE // CHUNK
    NJ2 = (nwin + NS - 1) // NS
    rem = nwin - (NJ2 - 1) * NS

    def valid2(s, j):
        return (j < NJ2 - 1) & (j >= 0) | ((j == NJ2 - 1) & (s < rem))

    def off2(s, j):
        j = jnp.clip(j, 0, NJ2 - 1)
        return (j * NS + s) * CHUNK

    def body(tA_hbm, tB_hbm, send_hbm, recv_hbm, outA_hbm, outB_hbm,
             tab_sh, idx_v, rows_v, semI, semG, semO):
        c = lax.axis_index("c")
        s = lax.axis_index("s")

        zbase = s * ROWS_PER_TILE

        def stage(tab_hbm):
            pltpu.sync_copy(tab_hbm.at[pl.ds(zbase, ROWS_PER_TILE)],
                            tab_sh.at[pl.ds(zbase, ROWS_PER_TILE)])

            @pl.when(s == NS - 1)
            def _():
                pltpu.sync_copy(tab_hbm.at[pl.ds(NS * ROWS_PER_TILE, 16)],
                                tab_sh.at[pl.ds(NS * ROWS_PER_TILE, 16)])

        @pl.when(c == 0)
        def _():
            stage(tA_hbm)

        @pl.when(c == 1)
        def _():
            stage(tB_hbm)

        plsc.subcore_barrier()

        def start_idx(idx_hbm, j):
            b = j % NB
            o = off2(s, j)
            pltpu.async_copy(idx_hbm.at[pl.ds(base + o, CHUNK)],
                             idx_v.at[b], semI)

        def wait_idx(idx_hbm, j):
            b = j % NB
            o = off2(s, j)
            pltpu.make_async_copy(idx_hbm.at[pl.ds(base + o, CHUNK)],
                                  idx_v.at[b], semI).wait()

        def start_gather(j):
            b = j % NB
            pltpu.async_copy(tab_sh.at[idx_v.at[b]], rows_v.at[b], semG)

        def wait_gather(j):
            b = j % NB
            pltpu.make_async_copy(tab_sh.at[idx_v.at[b]], rows_v.at[b],
                                  semG).wait()

        def start_out(out_hbm, j):
            b = j % NB
            o = off2(s, j)
            pltpu.async_copy(rows_v.at[b], out_hbm.at[pl.ds(o, CHUNK)],
                             semO)

        def wait_out(out_hbm, j):
            b = j % NB
            o = off2(s, j)
            pltpu.make_async_copy(rows_v.at[b],
                                  out_hbm.at[pl.ds(o, CHUNK)],
                                  semO).wait()

        def run(idx_hbm, out_hbm):
            start_idx(idx_hbm, 0)

            def step(j, _):
                @pl.when(valid2(s, j - NB))
                def _():
                    wait_out(out_hbm, j - NB)

                @pl.when(valid2(s, j))
                def _():
                    wait_idx(idx_hbm, j)
                    start_gather(j)

                @pl.when(valid2(s, j + 1))
                def _():
                    start_idx(idx_hbm, j + 1)

                @pl.when(valid2(s, j))
                def _():
                    wait_gather(j)
                    start_out(out_hbm, j)

                return 0

            lax.fori_loop(0, NJ2, step, 0)
            for dt in range(NB):
                t = NJ2 - 1 - dt

                @pl.when(valid2(s, t))
                def _():
                    wait_out(out_hbm, t)

        @pl.when(c == 0)
        def _():
            run(send_hbm, outA_hbm)

        @pl.when(c == 1)
        def _():
            run(recv_hbm, outB_hbm)

    f = pl.kernel(
        body,
        out_type=(jax.ShapeDtypeStruct((nE, D), jnp.float32),
                  jax.ShapeDtypeStruct((nE, D), jnp.float32)),
        mesh=_mesh(),
        scratch_types=[
            pltpu.VMEM_SHARED((N, D), jnp.float32),
            pltpu.VMEM((NB, CHUNK), jnp.int32),
            pltpu.VMEM((NB, CHUNK, D), jnp.float32),
            pltpu.SemaphoreType.DMA,
            pltpu.SemaphoreType.DMA,
            pltpu.SemaphoreType.DMA,
        ],
    )
    return f(tA, tB, send1d, recv1d)


def _layer_norm(h, gamma, beta):
    mu = jnp.mean(h, axis=-1, keepdims=True)
    var = jnp.mean((h - mu) ** 2, axis=-1, keepdims=True)
    return (h - mu) * lax.rsqrt(var + EPS) * gamma + beta


def _tc_node_mlp(nodes, agg0, agg1, nW1a, nW1b, nb1, nW2, nb2, ng, nbeta,
                 WA, WB):
    BN = 1000

    def body(nodes_ref, a0_ref, a1_ref, nW1a_ref, nW1b_ref, nb1_ref,
             nW2_ref, nb2_ref, ng_ref, nbeta_ref, WA_ref, WB_ref,
             nf_ref, tA_ref, tB_ref):
        x = nodes_ref[...]
        a = a0_ref[...] + a1_ref[...]
        h = (jnp.dot(x, nW1a_ref[...], preferred_element_type=jnp.float32)
             + jnp.dot(a, nW1b_ref[...], preferred_element_type=jnp.float32)
             + nb1_ref[...])
        h = jnp.dot(h, nW2_ref[...],
                    preferred_element_type=jnp.float32) + nb2_ref[...]
        nf = _layer_norm(h, ng_ref[...], nbeta_ref[...])
        nf_ref[...] = nf
        tA_ref[...] = jnp.dot(nf, WA_ref[...],
                              preferred_element_type=jnp.float32)
        tB_ref[...] = jnp.dot(nf, WB_ref[...],
                              preferred_element_type=jnp.float32)

    row_spec = pl.BlockSpec((BN, D), lambda i: (i, 0))
    w_spec = pl.BlockSpec((D, D), lambda i: (0, 0))
    v_spec = pl.BlockSpec((D,), lambda i: (0,))
    return pl.pallas_call(
        body,
        grid=(N // BN,),
        in_specs=[row_spec, row_spec, row_spec, w_spec, w_spec, v_spec,
                  w_spec, v_spec, v_spec, v_spec, w_spec, w_spec],
        out_specs=[row_spec, row_spec, row_spec],
        out_shape=[jax.ShapeDtypeStruct((N, D), jnp.float32)] * 3,
    )(nodes, agg0, agg1, nW1a, nW1b, nb1, nW2, nb2, ng, nbeta, WA, WB)


def _tc_edge_mlp_strip(prev, srcsA, dstsB, ef, Wc, bc, eg, ebeta,
                       block_off, n_rows):
    BE = 8000

    def body(*refs):
        if len(refs) == 8:
            sA_ref, dB_ref, ef_ref, Wc_ref, bc_ref, eg_ref, ebeta_ref, \
                out_ref = refs
        else:
            _, sA_ref, dB_ref, ef_ref, Wc_ref, bc_ref, eg_ref, \
                ebeta_ref, out_ref = refs
        h = (sA_ref[...] + dB_ref[...]
             + jnp.dot(ef_ref[...], Wc_ref[...],
                       preferred_element_type=jnp.float32) + bc_ref[...])
        out_ref[...] = _layer_norm(h, eg_ref[...], ebeta_ref[...])

    strip_spec = pl.BlockSpec((BE, D), lambda i: (i, 0))
    full_spec = pl.BlockSpec((BE, D), lambda i: (i + block_off, 0))
    w_spec = pl.BlockSpec((D, D), lambda i: (0, 0))
    v_spec = pl.BlockSpec((D,), lambda i: (0,))
    in_specs = [strip_spec, strip_spec, full_spec, w_spec, v_spec, v_spec,
                v_spec]
    args = (srcsA, dstsB, ef, Wc, bc, eg, ebeta)
    aliases = {}
    if prev is not None:
        in_specs = [pl.BlockSpec(memory_space=pl.ANY)] + in_specs
        args = (prev,) + args
        aliases = {0: 0}
    return pl.pallas_call(
        body,
        grid=(n_rows // BE,),
        in_specs=in_specs,
        out_specs=full_spec,
        out_shape=jax.ShapeDtypeStruct((E, D), jnp.float32),
        input_output_aliases=aliases,
    )(*args)


def kernel(node_features, edge_features, senders, receivers,
           nW1, nb1, nW2, nb2, ng, nbeta,
           eW1, eb1, eW2, eb2, eg, ebeta):
    nodes = node_features[0]
    ef = edge_features[0]

    WA = eW1[:D] @ eW2
    WB = eW1[D:2 * D] @ eW2
    Wc = eW1[2 * D:] @ eW2
    bc = eb1 @ eW2 + eb2

    recv2d = jnp.pad(receivers.reshape(E // CHUNK, CHUNK),
                     ((0, 12), (0, 0)))
    agg0, agg1 = _sc_segment_sum(recv2d, ef)
    nf, tA, tB = _tc_node_mlp(
        nodes, agg0, agg1, nW1[:D], nW1[D:], nb1, nW2, nb2, ng, nbeta,
        WA, WB)
    strips = (96000, 112000, 112000)
    gathered = []
    base = 0
    for nE in strips:
        gathered.append(_sc_gather2(tA, tB, senders, receivers, base, nE))
        base += nE
    ef_out = None
    base = 0
    for (sA, dB), nE in zip(gathered, strips):
        ef_out = _tc_edge_mlp_strip(ef_out, sA, dB, ef, Wc, bc, eg,
                                    ebeta, base // 8000, nE)
        base += nE
    return (nf[None], ef_out[None])
